# Initial kernel scaffold; baseline (speedup 1.0000x reference)
import jax
import jax.numpy as jnp
from jax.experimental import pallas as pl


def kernel(x_world, voxel_point, voxel_normal, score):
    raise NotImplementedError("write your pallas kernel here")


# speedup vs baseline: 34.4693x; 34.4693x over previous
import jax
import jax.numpy as jnp
from jax.experimental import pallas as pl
from jax.experimental.pallas import tpu as pltpu

Q = 1024
K = 16384
BQ = 64
GRID = Q // BQ


def _body(qx, qy, qz, px, py, pz, nx, ny, nz, sc,
          field_ref, nzj_ref, ss_scr, sn_scr):
    i = pl.program_id(0)

    qxv = qx[...]
    qyv = qy[...]
    qzv = qz[...]
    pxv = px[...]
    pyv = py[...]
    pzv = pz[...]
    nxv = nx[...]
    nyv = ny[...]
    nzv = nz[...]
    scv = sc[...]

    dx = qxv - pxv
    dy = qyv - pyv
    dz = qzv - pzv
    d2 = dx * dx + dy * dy + dz * dz
    d = jnp.sqrt(d2)

    m = jnp.full((BQ, 1), -jnp.inf, jnp.float32)
    for _ in range(8):
        masked = jnp.where(d > m, d, jnp.inf)
        m = jnp.min(masked, axis=1, keepdims=True)
    ind = (d <= m).astype(jnp.float32)

    cnt = jnp.sum(ind, axis=1, keepdims=True)
    sx = jnp.sum(ind * nxv, axis=1, keepdims=True)
    sy = jnp.sum(ind * nyv, axis=1, keepdims=True)
    sz = jnp.sum(ind * nzv, axis=1, keepdims=True)
    xn_x = sx / cnt
    xn_y = sy / cnt
    xn_z = sz / cnt

    na = jnp.sqrt(xn_x * xn_x + xn_y * xn_y + xn_z * xn_z)
    nb = jnp.sqrt(nxv * nxv + nyv * nyv + nzv * nzv)
    dot = xn_x * nxv + xn_y * nyv + xn_z * nzv
    thr = 0.75 * jnp.maximum(na * nb, 1e-6)
    mask = (dot > thr).astype(jnp.float32)

    sn_row = jnp.sum(mask, axis=1, keepdims=True)
    ss_row = jnp.sum(mask * (scv * jnp.exp(-d)), axis=1, keepdims=True)

    ss_scr[pl.ds(i * BQ, BQ), :] = ss_row
    sn_scr[pl.ds(i * BQ, BQ), :] = sn_row

    @pl.when(i == GRID - 1)
    def _finalize():
        ss_col = ss_scr[...]
        sn_col = sn_scr[...]
        nz_col = (sn_col != 0.0).astype(jnp.float32)
        val_col = ss_col / jnp.maximum(sn_col, 1.0)

        iota_p = jax.lax.broadcasted_iota(jnp.int32, (Q, Q), 0)
        iota_q = jax.lax.broadcasted_iota(jnp.int32, (Q, Q), 1)

        cum_row = jnp.sum(jnp.where(iota_p <= iota_q, nz_col, 0.0),
                          axis=0, keepdims=True)
        eye = (iota_p == iota_q)
        nz_row = jnp.sum(jnp.where(eye, nz_col, 0.0), axis=0, keepdims=True)
        val_row = jnp.sum(jnp.where(eye, val_col, 0.0), axis=0, keepdims=True)

        iota_j_f = iota_p.astype(jnp.float32)
        oh = jnp.where((cum_row - 1.0 == iota_j_f) & (nz_row != 0.0),
                       1.0, 0.0)
        nzj_col = jnp.sum(oh * iota_q.astype(jnp.float32),
                          axis=1, keepdims=True)
        rowsum = jnp.sum(oh, axis=1, keepdims=True)
        val0 = jax.lax.slice(val_col, (0, 0), (1, 1))
        field_col = (jnp.sum(oh * val_row, axis=1, keepdims=True)
                     + (1.0 - rowsum) * val0)

        field_ref[...] = field_col
        nzj_ref[...] = nzj_col.astype(jnp.int32)


def kernel(x_world, voxel_point, voxel_normal, score):
    q = x_world.reshape(Q, 3)
    p = voxel_point.reshape(K, 3)
    qx = q[:, 0].reshape(Q, 1)
    qy = q[:, 1].reshape(Q, 1)
    qz = q[:, 2].reshape(Q, 1)
    px = p[:, 0].reshape(1, K)
    py = p[:, 1].reshape(1, K)
    pz = p[:, 2].reshape(1, K)
    nx = voxel_normal[:, 0].reshape(1, K)
    ny = voxel_normal[:, 1].reshape(1, K)
    nz = voxel_normal[:, 2].reshape(1, K)
    sc = score.reshape(1, K)

    qspec = pl.BlockSpec((BQ, 1), lambda i: (i, 0))
    kspec = pl.BlockSpec((1, K), lambda i: (0, 0))
    ospec = pl.BlockSpec((Q, 1), lambda i: (0, 0))

    field, nzj = pl.pallas_call(
        _body,
        grid=(GRID,),
        in_specs=[qspec, qspec, qspec] + [kspec] * 7,
        out_specs=[ospec, ospec],
        out_shape=[
            jax.ShapeDtypeStruct((Q, 1), jnp.float32),
            jax.ShapeDtypeStruct((Q, 1), jnp.int32),
        ],
        scratch_shapes=[
            pltpu.VMEM((Q, 1), jnp.float32),
            pltpu.VMEM((Q, 1), jnp.float32),
        ],
    )(qx, qy, qz, px, py, pz, nx, ny, nz, sc)
    return field.reshape(Q), nzj.reshape(Q)


# speedup vs baseline: 34.9888x; 1.0151x over previous
import jax
import jax.numpy as jnp
from jax.experimental import pallas as pl
from jax.experimental.pallas import tpu as pltpu

Q = 1024
K = 16384
BQ = 128
GRID = Q // BQ


def _body(qx, qy, qz, px, py, pz, nx, ny, nz, sc,
          field_ref, nzj_ref, ss_scr, sn_scr):
    i = pl.program_id(0)

    qxv = qx[...]
    qyv = qy[...]
    qzv = qz[...]
    pxv = px[...]
    pyv = py[...]
    pzv = pz[...]
    nxv = nx[...]
    nyv = ny[...]
    nzv = nz[...]
    scv = sc[...]

    dx = qxv - pxv
    dy = qyv - pyv
    dz = qzv - pzv
    d2 = dx * dx + dy * dy + dz * dz
    d = jnp.sqrt(d2)

    m = jnp.full((BQ, 1), -jnp.inf, jnp.float32)
    for _ in range(8):
        masked = jnp.where(d > m, d, jnp.inf)
        m = jnp.min(masked, axis=1, keepdims=True)
    le = d <= m

    cnt = jnp.sum(jnp.where(le, 1.0, 0.0), axis=1, keepdims=True)
    sx = jnp.sum(jnp.where(le, nxv, 0.0), axis=1, keepdims=True)
    sy = jnp.sum(jnp.where(le, nyv, 0.0), axis=1, keepdims=True)
    sz = jnp.sum(jnp.where(le, nzv, 0.0), axis=1, keepdims=True)
    xn_x = sx / cnt
    xn_y = sy / cnt
    xn_z = sz / cnt

    na = jnp.sqrt(xn_x * xn_x + xn_y * xn_y + xn_z * xn_z)
    nb = jnp.sqrt(nxv * nxv + nyv * nyv + nzv * nzv)
    dot = xn_x * nxv + xn_y * nyv + xn_z * nzv
    thr = 0.75 * jnp.maximum(na * nb, 1e-6)
    gt = dot > thr

    sn_row = jnp.sum(jnp.where(gt, 1.0, 0.0), axis=1, keepdims=True)
    ss_row = jnp.sum(jnp.where(gt, scv * jnp.exp(-d), 0.0),
                     axis=1, keepdims=True)

    ss_scr[pl.ds(i * BQ, BQ), :] = ss_row
    sn_scr[pl.ds(i * BQ, BQ), :] = sn_row

    @pl.when(i == GRID - 1)
    def _finalize():
        ss_col = ss_scr[...]
        sn_col = sn_scr[...]
        nz_col = (sn_col != 0.0).astype(jnp.float32)
        val_col = ss_col / jnp.maximum(sn_col, 1.0)

        iota_p = jax.lax.broadcasted_iota(jnp.int32, (Q, Q), 0)
        iota_q = jax.lax.broadcasted_iota(jnp.int32, (Q, Q), 1)

        cum_row = jnp.sum(jnp.where(iota_p <= iota_q, nz_col, 0.0),
                          axis=0, keepdims=True)
        eye = (iota_p == iota_q)
        nz_row = jnp.sum(jnp.where(eye, nz_col, 0.0), axis=0, keepdims=True)
        val_row = jnp.sum(jnp.where(eye, val_col, 0.0), axis=0, keepdims=True)

        iota_j_f = iota_p.astype(jnp.float32)
        oh = jnp.where((cum_row - 1.0 == iota_j_f) & (nz_row != 0.0),
                       1.0, 0.0)
        nzj_col = jnp.sum(oh * iota_q.astype(jnp.float32),
                          axis=1, keepdims=True)
        rowsum = jnp.sum(oh, axis=1, keepdims=True)
        val0 = jax.lax.slice(val_col, (0, 0), (1, 1))
        field_col = (jnp.sum(oh * val_row, axis=1, keepdims=True)
                     + (1.0 - rowsum) * val0)

        field_ref[...] = field_col
        nzj_ref[...] = nzj_col.astype(jnp.int32)


def kernel(x_world, voxel_point, voxel_normal, score):
    q = x_world.reshape(Q, 3)
    p = voxel_point.reshape(K, 3)
    qx = q[:, 0].reshape(Q, 1)
    qy = q[:, 1].reshape(Q, 1)
    qz = q[:, 2].reshape(Q, 1)
    px = p[:, 0].reshape(1, K)
    py = p[:, 1].reshape(1, K)
    pz = p[:, 2].reshape(1, K)
    nx = voxel_normal[:, 0].reshape(1, K)
    ny = voxel_normal[:, 1].reshape(1, K)
    nz = voxel_normal[:, 2].reshape(1, K)
    sc = score.reshape(1, K)

    qspec = pl.BlockSpec((BQ, 1), lambda i: (i, 0))
    kspec = pl.BlockSpec((1, K), lambda i: (0, 0))
    ospec = pl.BlockSpec((Q, 1), lambda i: (0, 0))

    field, nzj = pl.pallas_call(
        _body,
        grid=(GRID,),
        in_specs=[qspec, qspec, qspec] + [kspec] * 7,
        out_specs=[ospec, ospec],
        out_shape=[
            jax.ShapeDtypeStruct((Q, 1), jnp.float32),
            jax.ShapeDtypeStruct((Q, 1), jnp.int32),
        ],
        scratch_shapes=[
            pltpu.VMEM((Q, 1), jnp.float32),
            pltpu.VMEM((Q, 1), jnp.float32),
        ],
    )(qx, qy, qz, px, py, pz, nx, ny, nz, sc)
    return field.reshape(Q), nzj.reshape(Q)
